# Initial kernel scaffold; baseline (speedup 1.0000x reference)
import jax
import jax.numpy as jnp
from jax.experimental import pallas as pl


def kernel(in_context_fc7, in_context_bb, in_bb_mask, in_context, in_c_mask, in_answer_fc7, in_answer_bb, in_answers, in_a_mask, emb_table):
    raise NotImplementedError("write your pallas kernel here")


# speedup vs baseline: 1.8906x; 1.8906x over previous
import functools

import jax
import jax.numpy as jnp
from jax.experimental import pallas as pl
from jax.experimental.pallas import tpu as pltpu
from jax.experimental.pallas import tpu_sc as plsc

D_WORD = 64
LANES = 16
N_CHUNKS = D_WORD // LANES
GCHUNK = 128
NW = 32

ANS_WINDOW = 256
SEG_LEN = 20
SEG_BLOCK = 32
CTX_WINDOW = SEG_BLOCK * SEG_LEN
CBLK = CTX_WINDOW // GCHUNK


def _emb_kernel(table, ans_idx, ctx_idx, mask_pad):
    n_ans = ans_idx.shape[0] * GCHUNK
    n_ctx = ctx_idx.shape[0] * GCHUNK
    n_seg = n_ctx // SEG_LEN
    steps = n_ctx // CTX_WINDOW // NW
    assert n_ans % (ANS_WINDOW * NW) == 0 and n_ctx % (CTX_WINDOW * NW) == 0

    mesh = plsc.VectorSubcoreMesh(core_axis_name="c", subcore_axis_name="s")

    @functools.partial(
        pl.kernel,
        out_type=(
            jax.ShapeDtypeStruct((n_ans, D_WORD), jnp.float32),
            jax.ShapeDtypeStruct((n_seg // 2, 2 * D_WORD), jnp.float32),
        ),
        mesh=mesh,
        scratch_types=[
            pltpu.VMEM((2, CTX_WINDOW, D_WORD), jnp.float32),
            pltpu.VMEM((2, CTX_WINDOW), jnp.int32),
            pltpu.SMEM((1,), jnp.int32),
            pltpu.SemaphoreType.DMA,
        ],
        compiler_params=pltpu.CompilerParams(use_tc_tiling_on_sc=False),
    )
    def k(aidx_hbm, cidx_hbm, mask_hbm, table_hbm, ans_hbm, ctx_hbm,
          rows_v, idx_v, cnt_ref, sem):
        def ans_body(i_vmem, o_vmem):
            for j in range(ANS_WINDOW // GCHUNK):
                pltpu.sync_copy(
                    table_hbm.at[i_vmem.at[j]],
                    o_vmem.at[pl.ds(j * GCHUNK, GCHUNK)],
                )

        pltpu.emit_pipeline(
            ans_body,
            grid=(n_ans // ANS_WINDOW,),
            in_specs=[pl.BlockSpec((ANS_WINDOW // GCHUNK, GCHUNK), lambda i: (i, 0))],
            out_specs=[pl.BlockSpec((ANS_WINDOW, D_WORD), lambda i: (i, 0))],
            core_axis_name=("c", "s"),
            dimension_semantics=(pltpu.PARALLEL,),
        )(aidx_hbm, ans_hbm)

        cnt_ref[0] = 0

        def gather_descs(src_idx, buf):
            return [
                pltpu.make_async_copy(
                    table_hbm.at[src_idx.at[pl.ds(j * GCHUNK, GCHUNK)]],
                    rows_v.at[buf, pl.ds(j * GCHUNK, GCHUNK)],
                    sem,
                )
                for j in range(CBLK)
            ]

        def ctx_body(iA, iB, m_vmem, o_vmem):
            c = cnt_ref[0]
            parity = jax.lax.rem(c, 2)
            first = c == 0
            last = c == steps - 1

            @pl.loop(0, CBLK)
            def _(j):
                for q in range(GCHUNK // LANES):
                    idx_v[1 - parity, pl.ds(j * GCHUNK + q * LANES, LANES)] = (
                        iB[j, pl.ds(q * LANES, LANES)]
                    )

            @pl.when(first)
            def _():
                @pl.loop(0, CBLK)
                def _(j):
                    for q in range(GCHUNK // LANES):
                        idx_v[0, pl.ds(j * GCHUNK + q * LANES, LANES)] = (
                            iA[j, pl.ds(q * LANES, LANES)]
                        )
                for d in gather_descs(idx_v.at[0], 0):
                    d.start()
                    d.wait()

            @pl.when(jnp.logical_not(first))
            def _():
                for d in gather_descs(idx_v.at[parity], parity):
                    d.wait()

            @pl.when(jnp.logical_not(last))
            def _():
                for d in gather_descs(idx_v.at[1 - parity], 1 - parity):
                    d.start()

            @pl.loop(0, SEG_BLOCK)
            def _(s):
                base = s * SEG_LEN
                m_vecs = [m_vmem[s, pl.ds(0, LANES)], m_vmem[s, pl.ds(LANES, LANES)]]
                accs = [jnp.zeros((LANES,), jnp.float32) for _ in range(N_CHUNKS)]
                for w in range(SEG_LEN):
                    m = m_vecs[w // LANES][w % LANES]
                    for ch in range(N_CHUNKS):
                        accs[ch] = accs[ch] + rows_v[parity, base + w,
                                                     pl.ds(ch * LANES, LANES)] * m
                half = jax.lax.rem(s, 2) * D_WORD
                for ch in range(N_CHUNKS):
                    o_vmem[jax.lax.div(s, 2),
                           pl.ds(half + ch * LANES, LANES)] = accs[ch]

            cnt_ref[0] = c + 1

        pltpu.emit_pipeline(
            ctx_body,
            grid=(NW, steps),
            in_specs=[
                pl.BlockSpec((CBLK, GCHUNK),
                             lambda w, t: (w * steps + t, 0)),
                pl.BlockSpec((CBLK, GCHUNK),
                             lambda w, t: (w * steps + jnp.minimum(t + 1, steps - 1), 0)),
                pl.BlockSpec((SEG_BLOCK, 2 * LANES),
                             lambda w, t: (w * steps + t, 0)),
            ],
            out_specs=[pl.BlockSpec((SEG_BLOCK // 2, 2 * D_WORD),
                                    lambda w, t: (w * steps + t, 0))],
            core_axis_name=("c", "s"),
            dimension_semantics=(pltpu.PARALLEL, pltpu.ARBITRARY),
        )(cidx_hbm, cidx_hbm, mask_hbm, ctx_hbm)

    return k(ans_idx, ctx_idx, mask_pad, table)


def kernel(in_context_fc7, in_context_bb, in_bb_mask, in_context, in_c_mask,
           in_answer_fc7, in_answer_bb, in_answers, in_a_mask, emb_table):
    B, A, W = in_answers.shape
    _, C, NB, _ = in_context.shape
    n_ans = B * A * W
    n_seg = B * C * NB

    ans_idx = in_answers.reshape(n_ans // GCHUNK, GCHUNK).astype(jnp.int32)
    ctx_idx = in_context.reshape(n_seg * W // GCHUNK, GCHUNK).astype(jnp.int32)
    ctx_mask = in_c_mask.reshape(n_seg, W)
    mask_pad = jnp.pad(ctx_mask, ((0, 0), (0, 2 * LANES - W)))

    answers_emb, context_box_rep = _emb_kernel(emb_table, ans_idx, ctx_idx, mask_pad)
    return (answers_emb.reshape(B, A, W, D_WORD),
            context_box_rep.reshape(B, C, NB, D_WORD))
